# baseline (device time: 31969 ns/iter reference)
import jax
import jax.numpy as jnp
from jax import lax
from jax.experimental import pallas as pl
from jax.experimental.pallas import tpu as pltpu

Y = 4
M_OUT = 128
D = 512
N = 2048


def kernel(x, dy):
    def body(x_ref, dy_ref, out_ref, part_ref, comm_ref, send_sems, recv_sems):
        my_x = lax.axis_index("x")
        my_y = lax.axis_index("y")
        my_z = lax.axis_index("z")
        right = lax.rem(my_y + 1, Y)
        left = lax.rem(my_y + Y - 1, Y)

        barrier_sem = pltpu.get_barrier_semaphore()
        for nbr in (left, right):
            pl.semaphore_signal(
                barrier_sem, inc=1,
                device_id=(my_x, nbr, my_z),
                device_id_type=pl.DeviceIdType.MESH,
            )
        pl.semaphore_wait(barrier_sem, 2)

        part = lax.dot_general(
            x_ref[...].astype(jnp.bfloat16),
            dy_ref[...].astype(jnp.bfloat16),
            dimension_numbers=(((0,), (0,)), ((), ())),
            preferred_element_type=jnp.float32,
        )
        part_ref[...] = part.astype(jnp.bfloat16)

        def chunk(j):
            return pl.ds(j * M_OUT, M_OUT)

        comm_ref[0, :, :] = part_ref[chunk(lax.rem(my_y + Y - 1, Y)), :]

        for s in range(Y - 1):
            rdma = pltpu.make_async_remote_copy(
                src_ref=comm_ref.at[s],
                dst_ref=comm_ref.at[s + 1],
                send_sem=send_sems.at[s],
                recv_sem=recv_sems.at[s],
                device_id=(my_x, right, my_z),
                device_id_type=pl.DeviceIdType.MESH,
            )
            rdma.start()
            rdma.wait()

            rc = lax.rem(my_y + 2 * Y - s - 2, Y)
            if s < Y - 2:
                comm_ref[s + 1, :, :] = (
                    comm_ref[s + 1, :, :] + part_ref[chunk(rc), :]
                )
            else:
                out_ref[...] = (
                    comm_ref[s + 1, :, :].astype(jnp.float32)
                    + part_ref[chunk(rc), :].astype(jnp.float32)
                )

    return pl.pallas_call(
        body,
        out_shape=jax.ShapeDtypeStruct((M_OUT, N), jnp.float32),
        in_specs=[
            pl.BlockSpec(memory_space=pltpu.VMEM),
            pl.BlockSpec(memory_space=pltpu.VMEM),
        ],
        out_specs=pl.BlockSpec(memory_space=pltpu.VMEM),
        scratch_shapes=[
            pltpu.VMEM((D, N), jnp.bfloat16),
            pltpu.VMEM((Y, M_OUT, N), jnp.bfloat16),
            pltpu.SemaphoreType.DMA((Y - 1,)),
            pltpu.SemaphoreType.DMA((Y - 1,)),
        ],
        compiler_params=pltpu.CompilerParams(collective_id=0),
    )(x, dy)


# device time: 31004 ns/iter; 1.0311x vs baseline; 1.0311x over previous
import jax
import jax.numpy as jnp
from jax import lax
from jax.experimental import pallas as pl
from jax.experimental.pallas import tpu as pltpu

Y = 4
M_OUT = 128
D = 512
N = 2048
NH = N // 2


def kernel(x, dy):
    def body(x_ref, dy_ref, out_ref, part_ref, comm_r, comm_l,
             send_r, recv_r, send_l, recv_l):
        my_x = lax.axis_index("x")
        my_y = lax.axis_index("y")
        my_z = lax.axis_index("z")
        right = lax.rem(my_y + 1, Y)
        left = lax.rem(my_y + Y - 1, Y)

        barrier_sem = pltpu.get_barrier_semaphore()
        for nbr in (left, right):
            pl.semaphore_signal(
                barrier_sem, inc=1,
                device_id=(my_x, nbr, my_z),
                device_id_type=pl.DeviceIdType.MESH,
            )
        pl.semaphore_wait(barrier_sem, 2)

        part = lax.dot_general(
            x_ref[...].astype(jnp.bfloat16),
            dy_ref[...].astype(jnp.bfloat16),
            dimension_numbers=(((0,), (0,)), ((), ())),
            preferred_element_type=jnp.float32,
        )
        part_ref[...] = part.astype(jnp.bfloat16)

        def rows(j):
            return pl.ds(j * M_OUT, M_OUT)

        cols_r = pl.ds(0, NH)
        cols_l = pl.ds(NH, NH)

        comm_r[0, :, :] = part_ref[rows(lax.rem(my_y + Y - 1, Y)), cols_r]
        comm_l[0, :, :] = part_ref[rows(lax.rem(my_y + 1, Y)), cols_l]

        rdmas = []
        for s in range(Y - 1):
            rdma_r = pltpu.make_async_remote_copy(
                src_ref=comm_r.at[s],
                dst_ref=comm_r.at[s + 1],
                send_sem=send_r.at[s],
                recv_sem=recv_r.at[s],
                device_id=(my_x, right, my_z),
                device_id_type=pl.DeviceIdType.MESH,
            )
            rdma_l = pltpu.make_async_remote_copy(
                src_ref=comm_l.at[s],
                dst_ref=comm_l.at[s + 1],
                send_sem=send_l.at[s],
                recv_sem=recv_l.at[s],
                device_id=(my_x, left, my_z),
                device_id_type=pl.DeviceIdType.MESH,
            )
            rdma_r.start()
            rdma_l.start()
            rdmas += [rdma_r, rdma_l]

            rdma_r.wait_recv()
            rdma_l.wait_recv()

            rc_r = lax.rem(my_y + 2 * Y - s - 2, Y)
            rc_l = lax.rem(my_y + s + 2, Y)
            if s < Y - 2:
                comm_r[s + 1, :, :] = (
                    comm_r[s + 1, :, :] + part_ref[rows(rc_r), cols_r]
                )
                comm_l[s + 1, :, :] = (
                    comm_l[s + 1, :, :] + part_ref[rows(rc_l), cols_l]
                )
            else:
                out_ref[:, cols_r] = (
                    comm_r[s + 1, :, :].astype(jnp.float32)
                    + part_ref[rows(rc_r), cols_r].astype(jnp.float32)
                )
                out_ref[:, cols_l] = (
                    comm_l[s + 1, :, :].astype(jnp.float32)
                    + part_ref[rows(rc_l), cols_l].astype(jnp.float32)
                )

        for rdma in rdmas:
            rdma.wait_send()

    return pl.pallas_call(
        body,
        out_shape=jax.ShapeDtypeStruct((M_OUT, N), jnp.float32),
        in_specs=[
            pl.BlockSpec(memory_space=pltpu.VMEM),
            pl.BlockSpec(memory_space=pltpu.VMEM),
        ],
        out_specs=pl.BlockSpec(memory_space=pltpu.VMEM),
        scratch_shapes=[
            pltpu.VMEM((D, N), jnp.bfloat16),
            pltpu.VMEM((Y, M_OUT, NH), jnp.bfloat16),
            pltpu.VMEM((Y, M_OUT, NH), jnp.bfloat16),
            pltpu.SemaphoreType.DMA((Y - 1,)),
            pltpu.SemaphoreType.DMA((Y - 1,)),
            pltpu.SemaphoreType.DMA((Y - 1,)),
            pltpu.SemaphoreType.DMA((Y - 1,)),
        ],
        compiler_params=pltpu.CompilerParams(collective_id=0),
    )(x, dy)


# device time: 22443 ns/iter; 1.4245x vs baseline; 1.3815x over previous
import jax
import jax.numpy as jnp
from jax import lax
from jax.experimental import pallas as pl
from jax.experimental.pallas import tpu as pltpu

Y = 4
Z = 4
M_OUT = 128
D = 512
N = 2048
CW = N // Z


def kernel(x, dy):
    def body(x_ref, dy_ref, out_ref, part_ref, rs_buf, ag_buf,
             rs_send, rs_recv, ag_send, ag_recv):
        my_x = lax.axis_index("x")
        my_y = lax.axis_index("y")
        my_z = lax.axis_index("z")

        barrier_sem = pltpu.get_barrier_semaphore()
        for d in range(1, Y):
            pl.semaphore_signal(
                barrier_sem, inc=1,
                device_id=(my_x, lax.rem(my_y + d, Y), my_z),
                device_id_type=pl.DeviceIdType.MESH,
            )
        for d in range(1, Z):
            pl.semaphore_signal(
                barrier_sem, inc=1,
                device_id=(my_x, my_y, lax.rem(my_z + d, Z)),
                device_id_type=pl.DeviceIdType.MESH,
            )
        pl.semaphore_wait(barrier_sem, (Y - 1) + (Z - 1))

        part = lax.dot_general(
            x_ref[...].astype(jnp.bfloat16),
            dy_ref[:, pl.ds(my_z * CW, CW)].astype(jnp.bfloat16),
            dimension_numbers=(((0,), (0,)), ((), ())),
            preferred_element_type=jnp.float32,
        )
        part_ref[...] = part.astype(jnp.bfloat16)

        def rows(j):
            return pl.ds(j * M_OUT, M_OUT)

        rdmas = []
        for d in range(1, Y):
            tgt = lax.rem(my_y + d, Y)
            rdma = pltpu.make_async_remote_copy(
                src_ref=part_ref.at[rows(tgt), :],
                dst_ref=rs_buf.at[d - 1],
                send_sem=rs_send.at[d - 1],
                recv_sem=rs_recv.at[d - 1],
                device_id=(my_x, tgt, my_z),
                device_id_type=pl.DeviceIdType.MESH,
            )
            rdma.start()
            rdmas.append(rdma)

        for rdma in rdmas:
            rdma.wait_recv()
        acc = (
            part_ref[rows(my_y), :].astype(jnp.float32)
            + rs_buf[0].astype(jnp.float32)
            + rs_buf[1].astype(jnp.float32)
            + rs_buf[2].astype(jnp.float32)
        )
        out_ref[:, pl.ds(my_z * CW, CW)] = acc
        ag_buf[Z - 1, :, :] = acc.astype(jnp.bfloat16)

        for d in range(1, Z):
            tgt = lax.rem(my_z + d, Z)
            rdma = pltpu.make_async_remote_copy(
                src_ref=ag_buf.at[Z - 1],
                dst_ref=ag_buf.at[d - 1],
                send_sem=ag_send.at[d - 1],
                recv_sem=ag_recv.at[d - 1],
                device_id=(my_x, my_y, tgt),
                device_id_type=pl.DeviceIdType.MESH,
            )
            rdma.start()
            rdmas.append(rdma)

        for d in range(1, Z):
            rdmas[Y - 1 + d - 1].wait_recv()
            src_z = lax.rem(my_z + Z - d, Z)
            out_ref[:, pl.ds(src_z * CW, CW)] = ag_buf[d - 1].astype(jnp.float32)

        for rdma in rdmas:
            rdma.wait_send()

    return pl.pallas_call(
        body,
        out_shape=jax.ShapeDtypeStruct((M_OUT, N), jnp.float32),
        in_specs=[
            pl.BlockSpec(memory_space=pltpu.VMEM),
            pl.BlockSpec(memory_space=pltpu.VMEM),
        ],
        out_specs=pl.BlockSpec(memory_space=pltpu.VMEM),
        scratch_shapes=[
            pltpu.VMEM((D, CW), jnp.bfloat16),
            pltpu.VMEM((Y - 1, M_OUT, CW), jnp.bfloat16),
            pltpu.VMEM((Z, M_OUT, CW), jnp.bfloat16),
            pltpu.SemaphoreType.DMA((Y - 1,)),
            pltpu.SemaphoreType.DMA((Y - 1,)),
            pltpu.SemaphoreType.DMA((Z - 1,)),
            pltpu.SemaphoreType.DMA((Z - 1,)),
        ],
        compiler_params=pltpu.CompilerParams(collective_id=0),
    )(x, dy)
